# baseline (speedup 1.0000x reference)
import functools

import jax
import jax.numpy as jnp
from jax import lax
from jax.experimental import pallas as pl
from jax.experimental.pallas import tpu as pltpu
from jax.experimental.pallas import tpu_sc as plsc

_B = 4
_K = 16384
_FEAT = 256
_TOKEN_DIM = 256
_T = 1024
_RPD = 3
_NREG = 27
_KT = 2048
_R = 128

_NC = 2
_NS = 16
_NW = _NC * _NS
_BPW = (_B * _T) // _NW


def _imp_body(f_ref, w1_ref, b1_ref, w2_ref, b2_ref, o_ref):
    f = f_ref[0]
    h = lax.dot_general(w1_ref[...], f, (((0,), (0,)), ((), ())),
                        preferred_element_type=jnp.float32)
    h = jnp.maximum(h + b1_ref[...], 0.0)
    y = lax.dot_general(w2_ref[...], h, (((0,), (0,)), ((), ())),
                        preferred_element_type=jnp.float32)
    y = y + b2_ref[0, 0]
    o_ref[0, 0] = jnp.maximum(y, 0.0) + jnp.log1p(jnp.exp(-jnp.abs(y)))


def _importance(features, W1, b1, W2, b2):
    nt = _K // _KT
    return pl.pallas_call(
        _imp_body,
        grid=(_B, nt),
        in_specs=[
            pl.BlockSpec((1, _FEAT, _KT), lambda b, t: (b, 0, t)),
            pl.BlockSpec((_FEAT, _FEAT // 2), lambda b, t: (0, 0)),
            pl.BlockSpec((_FEAT // 2, 1), lambda b, t: (0, 0)),
            pl.BlockSpec((_FEAT // 2, 1), lambda b, t: (0, 0)),
            pl.BlockSpec(memory_space=pltpu.SMEM),
        ],
        out_specs=pl.BlockSpec((1, 1, 1, _KT), lambda b, t: (b, t, 0, 0)),
        out_shape=jax.ShapeDtypeStruct((_B, nt, 1, _KT), jnp.float32),
    )(features, W1, b1.reshape(_FEAT // 2, 1), W2, b2.reshape(1, 1))


def _region_and_valid(xs, ys, zs):
    mnx, mxx = jnp.min(xs), jnp.max(xs)
    mny, mxy = jnp.min(ys), jnp.max(ys)
    mnz, mxz = jnp.min(zs), jnp.max(zs)
    xn = (xs - mnx) / (mxx - mnx + 1e-06)
    yn = (ys - mny) / (mxy - mny + 1e-06)
    zn = (zs - mnz) / (mxz - mnz + 1e-06)
    rix = jnp.clip(xn * _RPD, 0, _RPD - 1).astype(jnp.int32)
    riy = jnp.clip(yn * _RPD, 0, _RPD - 1).astype(jnp.int32)
    riz = jnp.clip(zn * _RPD, 0, _RPD - 1).astype(jnp.int32)
    region = rix * (_RPD * _RPD) + riy * _RPD + riz
    valid = ((jnp.abs(xs) + jnp.abs(ys)) + jnp.abs(zs)) > 0
    return region, valid


def _rank_rowmajor(m, ut, slt):
    mf = m.astype(jnp.float32)
    lanecum = lax.dot_general(mf, ut, (((1,), (0,)), ((), ())),
                              preferred_element_type=jnp.float32)
    rowtot = lanecum[:, _R - 1:_R]
    rowpre = lax.dot_general(slt, rowtot, (((1,), (0,)), ((), ())),
                             preferred_element_type=jnp.float32)
    return ((rowpre + lanecum) - 1).astype(jnp.int32)


def _stats_body(xyz_ref, imp_ref, rimp_ref, cnt_ref, off_ref, pos_ref):
    b = pl.program_id(0)
    xs, ys, zs = xyz_ref[0, 0], xyz_ref[0, 1], xyz_ref[0, 2]
    imp = imp_ref[0]
    region, valid = _region_and_valid(xs, ys, zs)
    ri = lax.broadcasted_iota(jnp.int32, (_R, _R), 0)
    ci = lax.broadcasted_iota(jnp.int32, (_R, _R), 1)
    ut = (ri <= ci).astype(jnp.float32)
    slt = (ri > ci).astype(jnp.float32)
    pos = jnp.zeros((_R, _R), jnp.int32)
    off = jnp.int32(0)
    for r in range(_NREG):
        m = (region == r) & valid
        rimp_ref[b, r] = jnp.sum(jnp.where(m, imp, 0.0))
        c = jnp.sum(m.astype(jnp.int32))
        cnt_ref[b, r] = c
        off_ref[b, r] = off
        pos = pos + jnp.where(m, off + _rank_rowmajor(m, ut, slt), 0)
        off = off + c
    inv = jnp.logical_not(valid)
    pos = pos + jnp.where(inv, off + _rank_rowmajor(inv, ut, slt), 0)
    pos_ref[0] = pos


def _region_stats(xyzT, impf):
    return pl.pallas_call(
        _stats_body,
        grid=(_B,),
        in_specs=[
            pl.BlockSpec((1, 3, _R, _R), lambda b: (b, 0, 0, 0)),
            pl.BlockSpec((1, _R, _R), lambda b: (b, 0, 0)),
        ],
        out_specs=[
            pl.BlockSpec(memory_space=pltpu.SMEM),
            pl.BlockSpec(memory_space=pltpu.SMEM),
            pl.BlockSpec(memory_space=pltpu.SMEM),
            pl.BlockSpec((1, _R, _R), lambda b: (b, 0, 0)),
        ],
        out_shape=[
            jax.ShapeDtypeStruct((_B, _NREG), jnp.float32),
            jax.ShapeDtypeStruct((_B, _NREG), jnp.int32),
            jax.ShapeDtypeStruct((_B, _NREG), jnp.int32),
            jax.ShapeDtypeStruct((_B, _R, _R), jnp.int32),
        ],
    )(xyzT, impf)


_RPW = (_B * _K) // _NW
_SCCH = _RPW // 128


def _scatter_rows(rows128, dpos):
    mesh = plsc.VectorSubcoreMesh(core_axis_name="c", subcore_axis_name="s")

    @functools.partial(
        pl.kernel,
        mesh=mesh,
        out_type=jax.ShapeDtypeStruct((_B * _K, 128), jnp.float32),
        scratch_types=[
            pltpu.VMEM((_SCCH, 128), jnp.int32),
            pltpu.VMEM((512, 128), jnp.float32),
            pltpu.SemaphoreType.DMA,
        ],
    )
    def k(rows_hbm, idx_hbm, out_hbm, idx_v, rows_v, sem):
        wid = lax.axis_index("s") * _NC + lax.axis_index("c")
        base = wid * _RPW
        pltpu.sync_copy(idx_hbm.at[wid], idx_v)
        for g in range(_RPW // 512):
            pltpu.sync_copy(rows_hbm.at[pl.ds(base + g * 512, 512)], rows_v)
            cps = [
                pltpu.async_copy(rows_v.at[pl.ds(j * 128, 128)],
                                 out_hbm.at[idx_v.at[g * 4 + j]], sem)
                for j in range(4)
            ]
            for cp in cps:
                cp.wait()

    return k(rows128, dpos)


def _select_body(sx_ref, ssm_ref, sidx_ref, off_ref, cnt_ref, take_ref,
                 ufps_ref, sel_ref, xyzsel_ref, scr_ref):
    b = pl.program_id(0)
    iota8 = (lax.broadcasted_iota(jnp.int32, (8, 128), 0) * 128
             + lax.broadcasted_iota(jnp.int32, (8, 128), 1))
    neg = jnp.float32(-jnp.inf)

    def emit(p2, j):
        sel_ref[b, p2] = sidx_ref[0, 0, j]
        cx = ssm_ref[0, 0, j]
        cy = ssm_ref[0, 1, j]
        cz = ssm_ref[0, 2, j]
        xyzsel_ref[b, 0, p2] = cx
        xyzsel_ref[b, 1, p2] = cy
        xyzsel_ref[b, 2, p2] = cz
        return cx, cy, cz

    def region_body(r, p):
        start = off_ref[b, r]
        cntr = cnt_ref[b, r]
        take = take_ref[b, r]
        fps_b = ufps_ref[b, r] > 0
        steps = jnp.minimum(take, _T - p)

        def fps_path(p):
            end = start + cntr
            c0 = start // 1024
            c1 = (end - 1) // 1024 + 1

            def initc(c, _):
                pv = c * 1024 + iota8
                m = (pv >= start) & (pv < end)
                scr_ref[pl.ds(c * 8, 8), :] = jnp.where(
                    m, jnp.float32(1e10), neg)
                return 0

            lax.fori_loop(c0, c1, initc, 0)

            def step(_, carry):
                p2, far = carry
                cx, cy, cz = emit(p2, far)

                def chunk(c, mcarry):
                    mx, pos = mcarry
                    rsl = pl.ds(c * 8, 8)
                    xs = sx_ref[0, 0, rsl, :]
                    ys = sx_ref[0, 1, rsl, :]
                    zs = sx_ref[0, 2, rsl, :]
                    dx = xs - cx
                    dy = ys - cy
                    dz = zs - cz
                    d = (dx * dx + dy * dy) + dz * dz
                    v = jnp.minimum(scr_ref[rsl, :], d)
                    scr_ref[rsl, :] = v
                    mx_c = jnp.max(v)
                    pv = c * 1024 + iota8
                    pos_c = jnp.min(jnp.where(v == mx_c, pv, jnp.int32(_K)))
                    better = mx_c > mx
                    return (jnp.where(better, mx_c, mx),
                            jnp.where(better, pos_c, pos))

                _, far2 = lax.fori_loop(c0, c1, chunk, (neg, jnp.int32(0)))
                return (p2 + 1, far2)

            return lax.fori_loop(0, steps, step, (p, start))[0]

        def ord_path(p):
            def step(s, p2):
                emit(p2, start + s)
                return p2 + 1

            return lax.fori_loop(0, steps, step, p)

        return lax.cond(fps_b, fps_path, ord_path, p)

    lax.fori_loop(0, _NREG, region_body, jnp.int32(0))


def _select(sxT, s3, sidx, off, cnt, take, ufps):
    return pl.pallas_call(
        _select_body,
        grid=(_B,),
        in_specs=[
            pl.BlockSpec((1, 3, _R, _R), lambda b: (b, 0, 0, 0)),
            pl.BlockSpec((1, 3, _K), lambda b: (b, 0, 0),
                         memory_space=pltpu.SMEM),
            pl.BlockSpec((1, 1, _K), lambda b: (b, 0, 0),
                         memory_space=pltpu.SMEM),
            pl.BlockSpec(memory_space=pltpu.SMEM),
            pl.BlockSpec(memory_space=pltpu.SMEM),
            pl.BlockSpec(memory_space=pltpu.SMEM),
            pl.BlockSpec(memory_space=pltpu.SMEM),
        ],
        out_specs=[
            pl.BlockSpec(memory_space=pltpu.SMEM),
            pl.BlockSpec(memory_space=pltpu.SMEM),
        ],
        out_shape=[
            jax.ShapeDtypeStruct((_B, _T), jnp.int32),
            jax.ShapeDtypeStruct((_B, 3, _T), jnp.float32),
        ],
        scratch_shapes=[pltpu.VMEM((_R, _R), jnp.float32)],
    )(sxT, s3, sidx, off, cnt, take, ufps)


def _gather_rows(table, gidx):
    mesh = plsc.VectorSubcoreMesh(core_axis_name="c", subcore_axis_name="s")

    @functools.partial(
        pl.kernel,
        mesh=mesh,
        out_type=jax.ShapeDtypeStruct((_B * _T, _FEAT), jnp.float32),
        scratch_types=[
            pltpu.VMEM((_BPW,), jnp.int32),
            pltpu.VMEM((_BPW, _FEAT), jnp.float32),
            pltpu.SemaphoreType.DMA,
        ],
    )
    def k(table_hbm, idx_hbm, out_hbm, idx_v, rows_v, sem):
        wid = lax.axis_index("s") * _NC + lax.axis_index("c")
        base = wid * _BPW
        pltpu.sync_copy(idx_hbm.at[pl.ds(base, _BPW)], idx_v)
        pltpu.async_copy(table_hbm.at[idx_v], rows_v, sem).wait()
        pltpu.sync_copy(rows_v, out_hbm.at[pl.ds(base, _BPW)])

    return k(table, gidx)


def _proj_body(rows_ref, wa_ref, ba_ref, len_ref, o_ref):
    b = pl.program_id(0)
    rows = rows_ref[0]
    fo = lax.dot_general(wa_ref[...], rows, (((0,), (1,)), ((), ())),
                         preferred_element_type=jnp.float32)
    fo = fo + ba_ref[...]
    tpos = lax.broadcasted_iota(jnp.int32, (1, _T), 1)
    o_ref[0] = jnp.where(tpos < len_ref[b, 0], fo, 0.0)


def _project(rows, Wa, ba, sel_len):
    return pl.pallas_call(
        _proj_body,
        grid=(_B,),
        in_specs=[
            pl.BlockSpec((1, _T, _FEAT), lambda b: (b, 0, 0)),
            pl.BlockSpec((_FEAT, _TOKEN_DIM), lambda b: (0, 0)),
            pl.BlockSpec((_TOKEN_DIM, 1), lambda b: (0, 0)),
            pl.BlockSpec(memory_space=pltpu.SMEM),
        ],
        out_specs=pl.BlockSpec((1, _TOKEN_DIM, _T), lambda b: (b, 0, 0)),
        out_shape=jax.ShapeDtypeStruct((_B, _TOKEN_DIM, _T), jnp.float32),
    )(rows, Wa, ba.reshape(_TOKEN_DIM, 1), sel_len.reshape(_B, 1))


def kernel(xyz, features, W1, b1, W2, b2, Wa, ba):
    xyzT = jnp.transpose(xyz, (0, 2, 1)).reshape(_B, 3, _R, _R)

    imp = _importance(features, W1, b1, W2, b2)
    impf = imp.reshape(_B, _R, _R)

    rimp, cnt, off, pos = _region_stats(xyzT, impf)

    idxf = lax.bitcast_convert_type(
        jnp.arange(_K, dtype=jnp.int32), jnp.float32)
    rows4 = jnp.concatenate(
        [xyz, jnp.broadcast_to(idxf[None, :, None], (_B, _K, 1))],
        axis=2).reshape(_B * _K, 4)
    rows128 = jnp.pad(rows4, ((0, 0), (0, 124)))
    dpos = (pos.reshape(_B, _K)
            + jnp.arange(_B, dtype=jnp.int32)[:, None] * _K)
    sorted128 = _scatter_rows(rows128, dpos.reshape(_NW, _SCCH, 128))
    s4 = sorted128[:, :4].reshape(_B, _K, 4)
    s3 = jnp.transpose(s4[..., :3], (0, 2, 1))
    sxT = s3.reshape(_B, 3, _R, _R)
    sidx = lax.bitcast_convert_type(
        s4[..., 3], jnp.int32).reshape(_B, 1, _K)

    n_rows, take_rows, len_rows = [], [], []
    for b in range(_B):
        total = rimp[b].sum() + 1e-08
        n_b = jnp.round(rimp[b] / total * _T).astype(jnp.int32)
        take_b = jnp.minimum(cnt[b], n_b)
        n_rows.append(n_b)
        take_rows.append(take_b)
        len_rows.append(jnp.minimum(jnp.sum(take_b), _T).astype(jnp.int32))
    n_r = jnp.stack(n_rows)
    take = jnp.stack(take_rows)
    sel_len = jnp.stack(len_rows)
    ufps = (cnt > n_r).astype(jnp.int32)

    sel, xyzsel = _select(sxT, s3, sidx, off, cnt, take, ufps)

    maskT = jnp.arange(_T)[None, :] < sel_len[:, None]
    xyz_out = jnp.where(maskT[:, :, None],
                        jnp.transpose(xyzsel, (0, 2, 1)), jnp.float32(0))
    sel_m = jnp.where(maskT, sel, 0)

    feat_rows = jnp.transpose(features, (0, 2, 1)).reshape(_B * _K, _FEAT)
    gidx = (sel_m + jnp.arange(_B, dtype=jnp.int32)[:, None] * _K)
    rows = _gather_rows(feat_rows, gidx.reshape(_B * _T))
    feat_out = _project(rows.reshape(_B, _T, _FEAT), Wa, ba, sel_len)
    return (xyz_out, feat_out)
